# Initial kernel scaffold; baseline (speedup 1.0000x reference)
import jax
import jax.numpy as jnp
from jax.experimental import pallas as pl


def kernel(x, table):
    raise NotImplementedError("write your pallas kernel here")


# speedup vs baseline: 2.9755x; 2.9755x over previous
import functools

import jax
import jax.numpy as jnp
from jax import lax
from jax.experimental import pallas as pl
from jax.experimental.pallas import tpu as pltpu
from jax.experimental.pallas import tpu_sc as plsc


@functools.cache
def _build_gather(B: int, D: int):
    info = plsc.get_sparse_core_info()
    NC, NS = info.num_cores, info.num_subcores
    NW = NC * NS
    assert B % NW == 0, (B, NW)
    bpw = B // NW
    CH = 128
    assert bpw % CH == 0, (bpw, CH)
    nch = bpw // CH

    mesh = plsc.VectorSubcoreMesh(core_axis_name="c", subcore_axis_name="s")

    def body(idx_hbm, tab_hbm, out_hbm, idx_v, rows_v, gsem):
        wid = lax.axis_index("s") * NC + lax.axis_index("c")
        base = wid * bpw
        pltpu.sync_copy(idx_hbm.at[pl.ds(base, bpw)], idx_v)

        def chunk(j, carry):
            pltpu.async_copy(
                tab_hbm.at[idx_v.at[pl.ds(j * CH, CH)]], rows_v, gsem
            ).wait()
            pltpu.sync_copy(rows_v, out_hbm.at[pl.ds(base + j * CH, CH)])
            return carry

        lax.fori_loop(0, nch, chunk, 0)

    return pl.kernel(
        body,
        out_type=jax.ShapeDtypeStruct((B, D), jnp.float32),
        mesh=mesh,
        scratch_types=[
            pltpu.VMEM((bpw,), jnp.int32),
            pltpu.VMEM((CH, D), jnp.float32),
            pltpu.SemaphoreType.DMA,
        ],
    )


def kernel(x, table):
    B = x.shape[0] * x.shape[1]
    idx = x.reshape(B).astype(jnp.int32)
    out = _build_gather(B, table.shape[1])(idx, table)
    return out.reshape(x.shape[0], x.shape[1], table.shape[1])


# speedup vs baseline: 3.3411x; 1.1229x over previous
import functools

import jax
import jax.numpy as jnp
from jax import lax
from jax.experimental import pallas as pl
from jax.experimental.pallas import tpu as pltpu
from jax.experimental.pallas import tpu_sc as plsc

CH = 128
NBUF = 4
K = 2


@functools.cache
def _build_gather(B: int, D: int):
    info = plsc.get_sparse_core_info()
    NC, NS = info.num_cores, info.num_subcores
    NW = NC * NS
    assert B % NW == 0, (B, NW)
    bpw = B // NW
    assert bpw % CH == 0, (bpw, CH)
    nch = bpw // CH
    head = NBUF
    tail = next(t for t in range(K, K + NBUF) if (nch - head - t) % NBUF == 0)
    assert nch >= head + tail

    mesh = plsc.VectorSubcoreMesh(core_axis_name="c", subcore_axis_name="s")

    def body(idx_hbm, tab_hbm, out_hbm, idx_v, bufs, gsems, ssems):
        wid = lax.axis_index("s") * NC + lax.axis_index("c")
        base = wid * bpw
        pltpu.sync_copy(idx_hbm.at[pl.ds(base, bpw)], idx_v)

        def g_desc(j, b):
            return pltpu.make_async_copy(
                tab_hbm.at[idx_v.at[pl.ds(j * CH, CH)]], bufs.at[b], gsems.at[b]
            )

        def s_desc(j, b):
            return pltpu.make_async_copy(
                bufs.at[b], out_hbm.at[pl.ds(base + j * CH, CH)], ssems.at[b]
            )

        def step(j, t, store_wait, next_gather):
            g_desc(j, t).wait()
            s_desc(j, t).start()
            if next_gather:
                b2 = (t + K) % NBUF
                if store_wait:
                    s_desc(j + K - NBUF, b2).wait()
                g_desc(j + K, b2).start()

        for j in range(K):
            g_desc(j, j % NBUF).start()
        for j in range(head):
            step(j, j % NBUF, j >= NBUF - K, j + K < nch)

        @pl.loop(head, nch - tail, step=NBUF)
        def _(j0):
            for t in range(NBUF):
                step(j0 + t, t, True, True)

        for j in range(nch - tail, nch):
            step(j, j % NBUF, j >= NBUF - K, j + K < nch)
        for j in range(nch - NBUF, nch):
            s_desc(j, j % NBUF).wait()

    return pl.kernel(
        body,
        out_type=jax.ShapeDtypeStruct((B, D), jnp.float32),
        mesh=mesh,
        scratch_types=[
            pltpu.VMEM((bpw,), jnp.int32),
            pltpu.VMEM((NBUF, CH, D), jnp.float32),
            pltpu.SemaphoreType.DMA((NBUF,)),
            pltpu.SemaphoreType.DMA((NBUF,)),
        ],
    )


def kernel(x, table):
    B = x.shape[0] * x.shape[1]
    idx = x.reshape(B).astype(jnp.int32)
    out = _build_gather(B, table.shape[1])(idx, table)
    return out.reshape(x.shape[0], x.shape[1], table.shape[1])


# speedup vs baseline: 3.3461x; 1.0015x over previous
import functools

import jax
import jax.numpy as jnp
from jax import lax
from jax.experimental import pallas as pl
from jax.experimental.pallas import tpu as pltpu
from jax.experimental.pallas import tpu_sc as plsc

CH = 128
NBUF = 7
K = 5


@functools.cache
def _build_gather(B: int, D: int):
    info = plsc.get_sparse_core_info()
    NC, NS = info.num_cores, info.num_subcores
    NW = NC * NS
    assert B % NW == 0, (B, NW)
    bpw = B // NW
    assert bpw % CH == 0, (bpw, CH)
    nch = bpw // CH
    head = NBUF
    tail = next(t for t in range(K, K + NBUF) if (nch - head - t) % NBUF == 0)
    assert nch >= head + tail

    mesh = plsc.VectorSubcoreMesh(core_axis_name="c", subcore_axis_name="s")

    def body(idx_hbm, tab_hbm, out_hbm, idx_v, bufs, gsems, ssems):
        wid = lax.axis_index("s") * NC + lax.axis_index("c")
        base = wid * bpw
        pltpu.sync_copy(idx_hbm.at[pl.ds(base, bpw)], idx_v)

        def g_desc(j, b):
            return pltpu.make_async_copy(
                tab_hbm.at[idx_v.at[pl.ds(j * CH, CH)]], bufs.at[b], gsems.at[b]
            )

        def s_desc(j, b):
            return pltpu.make_async_copy(
                bufs.at[b], out_hbm.at[pl.ds(base + j * CH, CH)], ssems.at[b]
            )

        def step(j, t, store_wait, next_gather):
            g_desc(j, t).wait()
            s_desc(j, t).start()
            if next_gather:
                b2 = (t + K) % NBUF
                if store_wait:
                    s_desc(j + K - NBUF, b2).wait()
                g_desc(j + K, b2).start()

        for j in range(K):
            g_desc(j, j % NBUF).start()
        for j in range(head):
            step(j, j % NBUF, j >= NBUF - K, j + K < nch)

        @pl.loop(head, nch - tail, step=NBUF)
        def _(j0):
            for t in range(NBUF):
                step(j0 + t, t, True, True)

        for j in range(nch - tail, nch):
            step(j, j % NBUF, j >= NBUF - K, j + K < nch)
        for j in range(nch - NBUF, nch):
            s_desc(j, j % NBUF).wait()

    return pl.kernel(
        body,
        out_type=jax.ShapeDtypeStruct((B, D), jnp.float32),
        mesh=mesh,
        scratch_types=[
            pltpu.VMEM((bpw,), jnp.int32),
            pltpu.VMEM((NBUF, CH, D), jnp.float32),
            pltpu.SemaphoreType.DMA((NBUF,)),
            pltpu.SemaphoreType.DMA((NBUF,)),
        ],
    )


def kernel(x, table):
    B = x.shape[0] * x.shape[1]
    idx = x.reshape(B).astype(jnp.int32)
    out = _build_gather(B, table.shape[1])(idx, table)
    return out.reshape(x.shape[0], x.shape[1], table.shape[1])


# speedup vs baseline: 5.9277x; 1.7715x over previous
import functools

import jax
import jax.numpy as jnp
from jax import lax
from jax.experimental import pallas as pl
from jax.experimental.pallas import tpu as pltpu
from jax.experimental.pallas import tpu_sc as plsc

JP = 56
NBUF = 8
K = 4


@functools.cache
def _build_gather(NI: int, JW: int, D: int):
    info = plsc.get_sparse_core_info()
    NC, NS = info.num_cores, info.num_subcores
    NW = NC * NS
    assert NI % NW == 0, (NI, NW)
    ipw = NI // NW
    nch = ipw
    head = NBUF
    tail = next(t for t in range(K, K + NBUF) if (nch - head - t) % NBUF == 0)
    assert nch >= head + tail

    mesh = plsc.VectorSubcoreMesh(core_axis_name="c", subcore_axis_name="s")

    def body(idx_hbm, tab_hbm, out_hbm, idx_v, bufs, gsems, ssems):
        wid = lax.axis_index("s") * NC + lax.axis_index("c")
        ibase = wid * ipw
        pltpu.sync_copy(idx_hbm.at[pl.ds(ibase * JP, ipw * JP)], idx_v)

        def g_desc(j, b):
            return pltpu.make_async_copy(
                tab_hbm.at[idx_v.at[pl.ds(j * JP, JW)]], bufs.at[b], gsems.at[b]
            )

        def s_desc(j, b):
            return pltpu.make_async_copy(
                bufs.at[b], out_hbm.at[ibase + j], ssems.at[b]
            )

        def step(j, t, store_wait, next_gather):
            g_desc(j, t).wait()
            s_desc(j, t).start()
            if next_gather:
                b2 = (t + K) % NBUF
                if store_wait:
                    s_desc(j + K - NBUF, b2).wait()
                g_desc(j + K, b2).start()

        for j in range(K):
            g_desc(j, j % NBUF).start()
        for j in range(head):
            step(j, j % NBUF, j >= NBUF - K, j + K < nch)

        @pl.loop(head, nch - tail, step=NBUF)
        def _(j0):
            for t in range(NBUF):
                step(j0 + t, t, True, True)

        for j in range(nch - tail, nch):
            step(j, j % NBUF, j >= NBUF - K, j + K < nch)
        for j in range(nch - NBUF, nch):
            s_desc(j, j % NBUF).wait()

    return pl.kernel(
        body,
        out_type=jax.ShapeDtypeStruct((NI, JW, D), jnp.float32),
        mesh=mesh,
        compiler_params=pltpu.CompilerParams(use_tc_tiling_on_sc=True),
        scratch_types=[
            pltpu.VMEM((ipw * JP,), jnp.int32),
            pltpu.VMEM((NBUF, JW, D), jnp.float32),
            pltpu.SemaphoreType.DMA((NBUF,)),
            pltpu.SemaphoreType.DMA((NBUF,)),
        ],
    )


def kernel(x, table):
    NI, JW = x.shape
    xi = x.astype(jnp.int32)
    idxp = jnp.pad(xi, ((0, 0), (0, JP - JW))).reshape(NI * JP)
    return _build_gather(NI, JW, table.shape[1])(idxp, table)


# speedup vs baseline: 10.5644x; 1.7822x over previous
import functools

import jax
import jax.numpy as jnp
from jax import lax
from jax.experimental import pallas as pl
from jax.experimental.pallas import tpu as pltpu
from jax.experimental.pallas import tpu_sc as plsc

CH = 128
NBUF = 7
K = 5


@functools.cache
def _build_gather(B: int, D: int):
    info = plsc.get_sparse_core_info()
    NC, NS = info.num_cores, info.num_subcores
    NW = NC * NS
    assert B % NW == 0, (B, NW)
    bpw = B // NW
    assert bpw % CH == 0, (bpw, CH)
    nch = bpw // CH
    head = NBUF
    tail = next(t for t in range(K, K + NBUF) if (nch - head - t) % NBUF == 0)
    assert nch >= head + tail

    mesh = plsc.VectorSubcoreMesh(core_axis_name="c", subcore_axis_name="s")

    def body(idx_hbm, tab_hbm, out_hbm, idx_v, bufs, gsems, ssems):
        wid = lax.axis_index("s") * NC + lax.axis_index("c")
        base = wid * bpw
        pltpu.sync_copy(idx_hbm.at[pl.ds(base, bpw)], idx_v)

        def g_desc(j, b):
            return pltpu.make_async_copy(
                tab_hbm.at[idx_v.at[pl.ds(j * CH, CH)]], bufs.at[b], gsems.at[b]
            )

        def s_desc(j, b):
            return pltpu.make_async_copy(
                bufs.at[b], out_hbm.at[pl.ds(base + j * CH, CH)], ssems.at[b]
            )

        def step(j, t, store_wait, next_gather):
            g_desc(j, t).wait()
            s_desc(j, t).start()
            if next_gather:
                b2 = (t + K) % NBUF
                if store_wait:
                    s_desc(j + K - NBUF, b2).wait()
                g_desc(j + K, b2).start()

        for j in range(K):
            g_desc(j, j % NBUF).start()
        for j in range(head):
            step(j, j % NBUF, j >= NBUF - K, j + K < nch)

        @pl.loop(head, nch - tail, step=NBUF)
        def _(j0):
            for t in range(NBUF):
                step(j0 + t, t, True, True)

        for j in range(nch - tail, nch):
            step(j, j % NBUF, j >= NBUF - K, j + K < nch)
        for j in range(nch - NBUF, nch):
            s_desc(j, j % NBUF).wait()

    return pl.kernel(
        body,
        out_type=jax.ShapeDtypeStruct((B, D), jnp.float32),
        mesh=mesh,
        scratch_types=[
            pltpu.VMEM((bpw,), jnp.int32),
            pltpu.VMEM((NBUF, CH, D), jnp.float32),
            pltpu.SemaphoreType.DMA((NBUF,)),
            pltpu.SemaphoreType.DMA((NBUF,)),
        ],
    )


def kernel(x, table):
    NI, JW = x.shape
    B = NI * JW
    idx = x.T.astype(jnp.int32).reshape(B)
    out2d = _build_gather(B, table.shape[1])(idx, table)
    return out2d.reshape(JW, NI, table.shape[1]).transpose(1, 0, 2)


# speedup vs baseline: 10.5765x; 1.0011x over previous
import functools

import jax
import jax.numpy as jnp
from jax import lax
from jax.experimental import pallas as pl
from jax.experimental.pallas import tpu as pltpu
from jax.experimental.pallas import tpu_sc as plsc

CH = 128
NBUF = 7
K = 5


@functools.cache
def _build_gather(B: int, D: int):
    info = plsc.get_sparse_core_info()
    NC, NS = info.num_cores, info.num_subcores
    NW = NC * NS
    assert B % NW == 0, (B, NW)
    bpw = B // NW
    assert bpw % CH == 0, (bpw, CH)
    nch = bpw // CH
    head = NBUF
    tail = next(t for t in range(K, K + NBUF) if (nch - head - t) % NBUF == 0)
    assert nch >= head + tail

    mesh = plsc.VectorSubcoreMesh(core_axis_name="c", subcore_axis_name="s")

    def body(idx_hbm, tab_hbm, out_hbm, idx_v, bufs, gsems, ssems):
        wid = lax.axis_index("s") * NC + lax.axis_index("c")
        base = wid * bpw
        pltpu.sync_copy(idx_hbm.at[pl.ds(base, bpw)], idx_v)

        def g_desc(j, b):
            return pltpu.make_async_copy(
                tab_hbm.at[idx_v.at[pl.ds(j * CH, CH)]], bufs.at[b], gsems.at[b]
            )

        def s_desc(j, b):
            return pltpu.make_async_copy(
                bufs.at[b], out_hbm.at[pl.ds(base + j * CH, CH)], ssems.at[b]
            )

        def step(j, t, store_wait, next_gather):
            g_desc(j, t).wait()
            s_desc(j, t).start()
            if next_gather:
                b2 = (t + K) % NBUF
                if store_wait:
                    s_desc(j + K - NBUF, b2).wait()
                g_desc(j + K, b2).start()

        for j in range(K):
            g_desc(j, j % NBUF).start()
        for j in range(head):
            step(j, j % NBUF, j >= NBUF - K, j + K < nch)

        @pl.loop(head, nch - tail, step=NBUF)
        def _(j0):
            for t in range(NBUF):
                step(j0 + t, t, True, True)

        for j in range(nch - tail, nch):
            step(j, j % NBUF, j >= NBUF - K, j + K < nch)
        for j in range(nch - NBUF, nch):
            s_desc(j, j % NBUF).wait()

    return pl.kernel(
        body,
        out_type=jax.ShapeDtypeStruct((B, D), jnp.float32),
        mesh=mesh,
        compiler_params=pltpu.CompilerParams(
            disable_bounds_checks=True,
            disable_semaphore_checks=True,
        ),
        scratch_types=[
            pltpu.VMEM((bpw,), jnp.int32),
            pltpu.VMEM((NBUF, CH, D), jnp.float32),
            pltpu.SemaphoreType.DMA((NBUF,)),
            pltpu.SemaphoreType.DMA((NBUF,)),
        ],
    )


def kernel(x, table):
    NI, JW = x.shape
    B = NI * JW
    idx = x.T.astype(jnp.int32).reshape(B)
    out2d = _build_gather(B, table.shape[1])(idx, table)
    return out2d.reshape(JW, NI, table.shape[1]).transpose(1, 0, 2)


# speedup vs baseline: 10.6017x; 1.0024x over previous
import functools

import jax
import jax.numpy as jnp
from jax import lax
from jax.experimental import pallas as pl
from jax.experimental.pallas import tpu as pltpu
from jax.experimental.pallas import tpu_sc as plsc

CH = 128
NBUF = 7
K = 5


@functools.cache
def _build_gather(B: int, D: int):
    info = plsc.get_sparse_core_info()
    NC, NS = info.num_cores, info.num_subcores
    NW = NC * NS
    assert B % NW == 0, (B, NW)
    bpw = B // NW
    assert bpw % CH == 0, (bpw, CH)
    nch = bpw // CH
    head = NBUF
    tail = next(t for t in range(K, K + NBUF) if (nch - head - t) % NBUF == 0)
    assert nch >= head + tail

    mesh = plsc.VectorSubcoreMesh(core_axis_name="c", subcore_axis_name="s")

    def body(idx_hbm, tab_hbm, out_hbm, idx_v, bufs, gsems, ssems):
        wid = lax.axis_index("s") * NC + lax.axis_index("c")
        base = wid * bpw
        pltpu.sync_copy(idx_hbm.at[pl.ds(base, bpw)], idx_v)

        def g_desc(j, b):
            return pltpu.make_async_copy(
                tab_hbm.at[idx_v.at[pl.ds(j * CH, CH)]], bufs.at[b], gsems.at[b]
            )

        def s_desc(j, b):
            return pltpu.make_async_copy(
                bufs.at[b], out_hbm.at[pl.ds(base + j * CH, CH)], ssems.at[b]
            )

        def step(j, t, store_wait, next_gather):
            g_desc(j, t).wait()
            s_desc(j, t).start()
            if next_gather:
                b2 = (t + K) % NBUF
                if store_wait:
                    s_desc(j + K - NBUF, b2).wait()
                g_desc(j + K, b2).start()

        for j in range(K):
            g_desc(j, j % NBUF).start()
        for j in range(head):
            step(j, j % NBUF, j >= NBUF - K, j + K < nch)

        @pl.loop(head, nch - tail, step=NBUF)
        def _(j0):
            for t in range(NBUF):
                step(j0 + t, t, True, True)

        for j in range(nch - tail, nch):
            step(j, j % NBUF, j >= NBUF - K, j + K < nch)
        for j in range(nch - NBUF, nch):
            s_desc(j, j % NBUF).wait()

    return pl.kernel(
        body,
        out_type=jax.ShapeDtypeStruct((B, D), jnp.float32),
        mesh=mesh,
        compiler_params=pltpu.CompilerParams(
            skip_device_barrier=True,
        ),
        scratch_types=[
            pltpu.VMEM((bpw,), jnp.int32),
            pltpu.VMEM((NBUF, CH, D), jnp.float32),
            pltpu.SemaphoreType.DMA((NBUF,)),
            pltpu.SemaphoreType.DMA((NBUF,)),
        ],
    )


def kernel(x, table):
    NI, JW = x.shape
    B = NI * JW
    idx = x.T.astype(jnp.int32).reshape(B)
    out2d = _build_gather(B, table.shape[1])(idx, table)
    return out2d.reshape(JW, NI, table.shape[1]).transpose(1, 0, 2)
